# baseline (speedup 1.0000x reference)
import jax
import jax.numpy as jnp
from jax.experimental import pallas as pl
from jax.experimental.pallas import tpu as pltpu
from jax.experimental.pallas import tpu_sc as plsc

D = 768
H = 12
HD = 64
E = 64
K = 2
FFN = 64
SCALE = 0.1
NEG = -1e30

_F32 = jnp.float32
_BF16 = jnp.bfloat16


def _ln_qkv_kernel(x_ref, lnw_ref, lnb_ref, w_ref, b_ref, o_ref):
    x = x_ref[0].astype(_F32)
    m = jnp.mean(x, axis=1, keepdims=True)
    v = jnp.mean((x - m) ** 2, axis=1, keepdims=True)
    xn = (x - m) / jnp.sqrt(v + 1e-5) * lnw_ref[...] + lnb_ref[...]
    y = jnp.dot(xn.astype(_BF16), w_ref[...], preferred_element_type=_F32)
    y = y + b_ref[...]
    o_ref[0] = y.astype(_BF16)


def _attn_kernel(q_ref, k_ref, v_ref, x_ref, w_ref, b_ref, rw_ref,
                 o_ref, l_ref):
    b = pl.program_id(0)
    i = pl.program_id(1)
    hp = pl.program_id(2)
    q2 = q_ref[0]
    k2 = k_ref[0]
    v2 = v_ref[0]
    CK = 128
    S_FULL = k2.shape[0]
    outs = []
    for h in range(2):
        q = q2[:, h * HD:(h + 1) * HD]
        acc = None
        den = None
        for c in range(S_FULL // CK):
            kc = k2[c * CK:(c + 1) * CK, h * HD:(h + 1) * HD]
            vc = v2[c * CK:(c + 1) * CK, h * HD:(h + 1) * HD]
            s = jax.lax.dot_general(q, kc, (((1,), (1,)), ((), ())),
                                    preferred_element_type=_F32)
            e = jnp.exp(s)
            d = jnp.sum(e, axis=1, keepdims=True)
            o = jnp.dot(e.astype(_BF16), vc, preferred_element_type=_F32)
            acc = o if acc is None else acc + o
            den = d if den is None else den + d
        outs.append((acc * (1.0 / den)).astype(_BF16))
    o2 = jnp.concatenate(outs, axis=1)
    po = jnp.dot(o2, w_ref[...], preferred_element_type=_F32)

    @pl.when(hp == 0)
    def _():
        o_ref[0] = x_ref[0] + b_ref[...]

    o_ref[0] += po

    @pl.when(jnp.logical_and(i == 0, hp == H // 2 - 1))
    def _():
        row0 = o_ref[0][0:1, :]
        lb = jnp.dot(row0.astype(_BF16), rw_ref[...].astype(_BF16),
                     preferred_element_type=_F32)
        rows = jax.lax.broadcasted_iota(jnp.int32, (8, E), 0)

        @pl.when(b == 0)
        def _():
            l_ref[...] = jnp.where(rows == 0, lb, 0.0)

        @pl.when(b != 0)
        def _():
            l_ref[...] = jnp.where(rows == 1, lb, l_ref[...])


def _sc_router_body(l_hbm, idx_hbm, val_hbm, lbuf, ibuf, vbuf, sem):
    core = jax.lax.axis_index("core")

    @pl.when(core == 0)
    def _():
        pltpu.async_copy(l_hbm, lbuf, sem).wait()
        for b in range(2):
            vbuf[b, 0] = lbuf[b, 0]
            ibuf[b, 0] = 0

            @pl.loop(1, E)
            def _(i):
                v = lbuf[b, i]

                @pl.when(v > vbuf[b, 0])
                def _():
                    vbuf[b, 0] = v
                    ibuf[b, 0] = i

            vbuf[b, 1] = NEG
            ibuf[b, 1] = 0

            @pl.loop(0, E)
            def _(i):
                v = lbuf[b, i]

                @pl.when(jnp.logical_and(i != ibuf[b, 0], v > vbuf[b, 1]))
                def _():
                    vbuf[b, 1] = v
                    ibuf[b, 1] = i

        pltpu.async_copy(ibuf, idx_hbm, sem).wait()
        pltpu.async_copy(vbuf, val_hbm, sem).wait()


def _mlp_moe_kernel(idx_ref, g_ref, x_ref, lnw_ref, lnb_ref, wfc_ref, bfc_ref,
                    wpr_ref, bpr_ref, dw_ref, db_ref, uw_ref, ub_ref, o_ref):
    b = pl.program_id(0)
    k = pl.program_id(2)
    x = x_ref[0]

    @pl.when(k == 0)
    def _():
        m = jnp.mean(x, axis=1, keepdims=True)
        v = jnp.mean((x - m) ** 2, axis=1, keepdims=True)
        xn = (x - m) / jnp.sqrt(v + 1e-5) * lnw_ref[...] + lnb_ref[...]
        h = jnp.dot(xn.astype(_BF16), wfc_ref[...],
                    preferred_element_type=_F32)
        h = h + bfc_ref[...]
        h = h * jax.nn.sigmoid(1.702 * h)
        y = jnp.dot(h.astype(_BF16), wpr_ref[...],
                    preferred_element_type=_F32)
        o_ref[0] = y + bpr_ref[...] + x

    l1 = g_ref[b, 0]
    l2 = g_ref[b, 1]
    g1 = 1.0 / (1.0 + jnp.exp(l2 - l1))
    g = jnp.where(k == 0, g1, 1.0 - g1) * SCALE
    xh = x.astype(_BF16)
    hh = jnp.dot(xh, dw_ref[0].astype(_BF16),
                 preferred_element_type=_F32) + db_ref[0]
    hh = jnp.maximum(hh, 0.0)
    up = jnp.dot(hh.astype(_BF16), uw_ref[0].astype(_BF16),
                 preferred_element_type=_F32)
    o_ref[0] += g * (up + ub_ref[0])


def kernel(x, in_proj_w, in_proj_b, out_proj_w, out_proj_b, ln1_w, ln1_b,
           ln2_w, ln2_b, c_fc_w, c_fc_b, c_proj_w, c_proj_b, router,
           down_w, down_b, up_w, up_b):
    S, B, _ = x.shape
    BS = 1024
    BQ = 2048
    nS = S // BS

    xb = jnp.transpose(x, (1, 0, 2))
    qscale = jnp.concatenate([jnp.full((D,), 0.125, _F32),
                              jnp.ones((2 * D,), _F32)])
    w_in = (in_proj_w.T * qscale).astype(_BF16)
    in_proj_b = in_proj_b * qscale
    w_out = out_proj_w.T.astype(_BF16)
    w_fc = c_fc_w.T.astype(_BF16)
    w_pr = c_proj_w.T.astype(_BF16)
    db2 = down_b.reshape(E, 1, FFN)
    ub2 = up_b.reshape(E, 1, D)

    seq = ("arbitrary",)

    qkv = pl.pallas_call(
        _ln_qkv_kernel,
        grid=(B, nS),
        in_specs=[
            pl.BlockSpec((1, BS, D), lambda b, i: (b, i, 0)),
            pl.BlockSpec((1, D), lambda b, i: (0, 0)),
            pl.BlockSpec((1, D), lambda b, i: (0, 0)),
            pl.BlockSpec((D, 3 * D), lambda b, i: (0, 0)),
            pl.BlockSpec((1, 3 * D), lambda b, i: (0, 0)),
        ],
        out_specs=pl.BlockSpec((1, BS, 3 * D), lambda b, i: (b, i, 0)),
        out_shape=jax.ShapeDtypeStruct((B, S, 3 * D), _BF16),
        compiler_params=pltpu.CompilerParams(
            dimension_semantics=seq * 2),
    )(xb, ln1_w.reshape(1, D), ln1_b.reshape(1, D), w_in,
      in_proj_b.reshape(1, 3 * D))

    HP = H // 2
    x1, logits = pl.pallas_call(
        _attn_kernel,
        grid=(B, S // BQ, HP),
        in_specs=[
            pl.BlockSpec((1, BQ, 2 * HD), lambda b, i, h: (b, i, h)),
            pl.BlockSpec((1, S, 2 * HD), lambda b, i, h: (b, 0, HP + h)),
            pl.BlockSpec((1, S, 2 * HD), lambda b, i, h: (b, 0, 2 * HP + h)),
            pl.BlockSpec((1, BQ, D), lambda b, i, h: (b, i, 0)),
            pl.BlockSpec((2 * HD, D), lambda b, i, h: (h, 0)),
            pl.BlockSpec((1, D), lambda b, i, h: (0, 0)),
            pl.BlockSpec((D, E), lambda b, i, h: (0, 0)),
        ],
        out_specs=[
            pl.BlockSpec((1, BQ, D), lambda b, i, h: (b, i, 0)),
            pl.BlockSpec((8, E), lambda b, i, h: (0, 0)),
        ],
        out_shape=[
            jax.ShapeDtypeStruct((B, S, D), _F32),
            jax.ShapeDtypeStruct((8, E), _F32),
        ],
        compiler_params=pltpu.CompilerParams(
            dimension_semantics=seq * 3),
    )(qkv, qkv, qkv, xb, w_out, out_proj_b.reshape(1, D), router)

    sc_mesh = plsc.ScalarSubcoreMesh(axis_name="core", num_cores=2)
    idx_p, gate_p = pl.kernel(
        _sc_router_body,
        out_type=[
            jax.ShapeDtypeStruct((2, 16), jnp.int32),
            jax.ShapeDtypeStruct((2, 16), _F32),
        ],
        mesh=sc_mesh,
        scratch_types=[
            pltpu.SMEM((8, E), _F32),
            pltpu.SMEM((2, 16), jnp.int32),
            pltpu.SMEM((2, 16), _F32),
            pltpu.SemaphoreType.DMA,
        ],
    )(logits)

    grid_spec = pltpu.PrefetchScalarGridSpec(
        num_scalar_prefetch=2,
        grid=(B, nS, K),
        in_specs=[
            pl.BlockSpec((1, BS, D), lambda b, i, k, ir, gr: (b, i, 0)),
            pl.BlockSpec((1, D), lambda b, i, k, ir, gr: (0, 0)),
            pl.BlockSpec((1, D), lambda b, i, k, ir, gr: (0, 0)),
            pl.BlockSpec((D, 4 * D), lambda b, i, k, ir, gr: (0, 0)),
            pl.BlockSpec((1, 4 * D), lambda b, i, k, ir, gr: (0, 0)),
            pl.BlockSpec((4 * D, D), lambda b, i, k, ir, gr: (0, 0)),
            pl.BlockSpec((1, D), lambda b, i, k, ir, gr: (0, 0)),
            pl.BlockSpec((1, D, FFN),
                         lambda b, i, k, ir, gr: (ir[b, k], 0, 0)),
            pl.BlockSpec((1, 1, FFN),
                         lambda b, i, k, ir, gr: (ir[b, k], 0, 0)),
            pl.BlockSpec((1, FFN, D),
                         lambda b, i, k, ir, gr: (ir[b, k], 0, 0)),
            pl.BlockSpec((1, 1, D),
                         lambda b, i, k, ir, gr: (ir[b, k], 0, 0)),
        ],
        out_specs=pl.BlockSpec(
            (1, BS, D), lambda b, i, k, ir, gr: (b, i, 0)),
    )
    out_b = pl.pallas_call(
        _mlp_moe_kernel,
        grid_spec=grid_spec,
        out_shape=jax.ShapeDtypeStruct((B, S, D), _F32),
        compiler_params=pltpu.CompilerParams(
            dimension_semantics=seq * 3),
    )(idx_p, gate_p, x1, ln2_w.reshape(1, D), ln2_b.reshape(1, D), w_fc,
      c_fc_b.reshape(1, 4 * D), w_pr, c_proj_b.reshape(1, D),
      down_w, db2, up_w, ub2)

    return jnp.transpose(out_b, (1, 0, 2))


# speedup vs baseline: 1.0721x; 1.0721x over previous
import jax
import jax.numpy as jnp
from jax.experimental import pallas as pl
from jax.experimental.pallas import tpu as pltpu
from jax.experimental.pallas import tpu_sc as plsc

D = 768
H = 12
HD = 64
E = 64
K = 2
FFN = 64
SCALE = 0.1
NEG = -1e30

_F32 = jnp.float32
_BF16 = jnp.bfloat16


def _ln_qkv_kernel(x_ref, lnw_ref, lnb_ref, w_ref, b_ref, o_ref):
    x = x_ref[0].astype(_F32)
    m = jnp.mean(x, axis=1, keepdims=True)
    v = jnp.mean((x - m) ** 2, axis=1, keepdims=True)
    xn = (x - m) / jnp.sqrt(v + 1e-5) * lnw_ref[...] + lnb_ref[...]
    y = jnp.dot(xn.astype(_BF16), w_ref[...], preferred_element_type=_F32)
    y = y + b_ref[...]
    o_ref[0] = y.astype(_BF16)


def _attn_kernel(q_ref, k_ref, v_ref, x_ref, w_ref, b_ref, rw_ref,
                 o_ref, l_ref):
    b = pl.program_id(0)
    i = pl.program_id(1)
    hp = pl.program_id(2)
    q2 = q_ref[0]
    k2 = k_ref[0]
    v2 = v_ref[0]
    CK = 256
    S_FULL = k2.shape[0]
    outs = []
    for h in range(2):
        q = q2[:, h * HD:(h + 1) * HD]
        acc = None
        den = None
        for c in range(S_FULL // CK):
            kc = k2[c * CK:(c + 1) * CK, h * HD:(h + 1) * HD]
            vc = v2[c * CK:(c + 1) * CK, h * HD:(h + 1) * HD]
            s = jax.lax.dot_general(q, kc, (((1,), (1,)), ((), ())),
                                    preferred_element_type=_F32)
            e = jnp.exp(s)
            d = jnp.sum(e, axis=1, keepdims=True)
            o = jnp.dot(e.astype(_BF16), vc, preferred_element_type=_F32)
            acc = o if acc is None else acc + o
            den = d if den is None else den + d
        outs.append((acc * (1.0 / den)).astype(_BF16))
    o2 = jnp.concatenate(outs, axis=1)
    po = jnp.dot(o2, w_ref[...], preferred_element_type=_F32)

    @pl.when(hp == 0)
    def _():
        o_ref[0] = x_ref[0] + b_ref[...]

    o_ref[0] += po

    @pl.when(jnp.logical_and(i == 0, hp == H // 2 - 1))
    def _():
        row0 = o_ref[0][0:1, :]
        lb = jnp.dot(row0.astype(_BF16), rw_ref[...].astype(_BF16),
                     preferred_element_type=_F32)
        rows = jax.lax.broadcasted_iota(jnp.int32, (8, E), 0)

        @pl.when(b == 0)
        def _():
            l_ref[...] = jnp.where(rows == 0, lb, 0.0)

        @pl.when(b != 0)
        def _():
            l_ref[...] = jnp.where(rows == 1, lb, l_ref[...])


def _sc_router_body(l_hbm, idx_hbm, val_hbm, lbuf, ibuf, vbuf, sem):
    core = jax.lax.axis_index("core")

    @pl.when(core == 0)
    def _():
        pltpu.async_copy(l_hbm, lbuf, sem).wait()
        for b in range(2):
            vbuf[b, 0] = lbuf[b, 0]
            ibuf[b, 0] = 0

            @pl.loop(1, E)
            def _(i):
                v = lbuf[b, i]

                @pl.when(v > vbuf[b, 0])
                def _():
                    vbuf[b, 0] = v
                    ibuf[b, 0] = i

            vbuf[b, 1] = NEG
            ibuf[b, 1] = 0

            @pl.loop(0, E)
            def _(i):
                v = lbuf[b, i]

                @pl.when(jnp.logical_and(i != ibuf[b, 0], v > vbuf[b, 1]))
                def _():
                    vbuf[b, 1] = v
                    ibuf[b, 1] = i

        pltpu.async_copy(ibuf, idx_hbm, sem).wait()
        pltpu.async_copy(vbuf, val_hbm, sem).wait()


def _mlp_moe_kernel(idx_ref, g_ref, x_ref, lnw_ref, lnb_ref, wfc_ref, bfc_ref,
                    wpr_ref, bpr_ref, dw_ref, db_ref, uw_ref, ub_ref, o_ref):
    b = pl.program_id(0)
    k = pl.program_id(2)
    x = x_ref[0]

    @pl.when(k == 0)
    def _():
        m = jnp.mean(x, axis=1, keepdims=True)
        v = jnp.mean((x - m) ** 2, axis=1, keepdims=True)
        xn = (x - m) / jnp.sqrt(v + 1e-5) * lnw_ref[...] + lnb_ref[...]
        h = jnp.dot(xn.astype(_BF16), wfc_ref[...],
                    preferred_element_type=_F32)
        h = h + bfc_ref[...]
        h = h * jax.nn.sigmoid(1.702 * h)
        y = jnp.dot(h.astype(_BF16), wpr_ref[...],
                    preferred_element_type=_F32)
        o_ref[0] = y + bpr_ref[...] + x

    l1 = g_ref[b, 0]
    l2 = g_ref[b, 1]
    g1 = 1.0 / (1.0 + jnp.exp(l2 - l1))
    g = jnp.where(k == 0, g1, 1.0 - g1) * SCALE
    xh = x.astype(_BF16)
    hh = jnp.dot(xh, dw_ref[0].astype(_BF16),
                 preferred_element_type=_F32) + db_ref[0]
    hh = jnp.maximum(hh, 0.0)
    up = jnp.dot(hh.astype(_BF16), uw_ref[0].astype(_BF16),
                 preferred_element_type=_F32)
    o_ref[0] += g * (up + ub_ref[0])


def kernel(x, in_proj_w, in_proj_b, out_proj_w, out_proj_b, ln1_w, ln1_b,
           ln2_w, ln2_b, c_fc_w, c_fc_b, c_proj_w, c_proj_b, router,
           down_w, down_b, up_w, up_b):
    S, B, _ = x.shape
    BS = 512
    BQ = 2048
    nS = S // BS

    xb = jnp.transpose(x, (1, 0, 2))
    qscale = jnp.concatenate([jnp.full((D,), 0.125, _F32),
                              jnp.ones((2 * D,), _F32)])
    w_in = (in_proj_w.T * qscale).astype(_BF16)
    in_proj_b = in_proj_b * qscale
    w_out = out_proj_w.T.astype(_BF16)
    w_fc = c_fc_w.T.astype(_BF16)
    w_pr = c_proj_w.T.astype(_BF16)
    db2 = down_b.reshape(E, 1, FFN)
    ub2 = up_b.reshape(E, 1, D)

    seq = ("arbitrary",)

    qkv = pl.pallas_call(
        _ln_qkv_kernel,
        grid=(B, nS),
        in_specs=[
            pl.BlockSpec((1, BS, D), lambda b, i: (b, i, 0)),
            pl.BlockSpec((1, D), lambda b, i: (0, 0)),
            pl.BlockSpec((1, D), lambda b, i: (0, 0)),
            pl.BlockSpec((D, 3 * D), lambda b, i: (0, 0)),
            pl.BlockSpec((1, 3 * D), lambda b, i: (0, 0)),
        ],
        out_specs=pl.BlockSpec((1, BS, 3 * D), lambda b, i: (b, i, 0)),
        out_shape=jax.ShapeDtypeStruct((B, S, 3 * D), _BF16),
        compiler_params=pltpu.CompilerParams(
            dimension_semantics=seq * 2),
    )(xb, ln1_w.reshape(1, D), ln1_b.reshape(1, D), w_in,
      in_proj_b.reshape(1, 3 * D))

    HP = H // 2
    x1, logits = pl.pallas_call(
        _attn_kernel,
        grid=(B, S // BQ, HP),
        in_specs=[
            pl.BlockSpec((1, BQ, 2 * HD), lambda b, i, h: (b, i, h)),
            pl.BlockSpec((1, S, 2 * HD), lambda b, i, h: (b, 0, HP + h)),
            pl.BlockSpec((1, S, 2 * HD), lambda b, i, h: (b, 0, 2 * HP + h)),
            pl.BlockSpec((1, BQ, D), lambda b, i, h: (b, i, 0)),
            pl.BlockSpec((2 * HD, D), lambda b, i, h: (h, 0)),
            pl.BlockSpec((1, D), lambda b, i, h: (0, 0)),
            pl.BlockSpec((D, E), lambda b, i, h: (0, 0)),
        ],
        out_specs=[
            pl.BlockSpec((1, BQ, D), lambda b, i, h: (b, i, 0)),
            pl.BlockSpec((8, E), lambda b, i, h: (0, 0)),
        ],
        out_shape=[
            jax.ShapeDtypeStruct((B, S, D), _F32),
            jax.ShapeDtypeStruct((8, E), _F32),
        ],
        compiler_params=pltpu.CompilerParams(
            dimension_semantics=seq * 3),
    )(qkv, qkv, qkv, xb, w_out, out_proj_b.reshape(1, D), router)

    sc_mesh = plsc.ScalarSubcoreMesh(axis_name="core", num_cores=2)
    idx_p, gate_p = pl.kernel(
        _sc_router_body,
        out_type=[
            jax.ShapeDtypeStruct((2, 16), jnp.int32),
            jax.ShapeDtypeStruct((2, 16), _F32),
        ],
        mesh=sc_mesh,
        scratch_types=[
            pltpu.SMEM((8, E), _F32),
            pltpu.SMEM((2, 16), jnp.int32),
            pltpu.SMEM((2, 16), _F32),
            pltpu.SemaphoreType.DMA,
        ],
    )(logits)

    grid_spec = pltpu.PrefetchScalarGridSpec(
        num_scalar_prefetch=2,
        grid=(B, nS, K),
        in_specs=[
            pl.BlockSpec((1, BS, D), lambda b, i, k, ir, gr: (b, i, 0)),
            pl.BlockSpec((1, D), lambda b, i, k, ir, gr: (0, 0)),
            pl.BlockSpec((1, D), lambda b, i, k, ir, gr: (0, 0)),
            pl.BlockSpec((D, 4 * D), lambda b, i, k, ir, gr: (0, 0)),
            pl.BlockSpec((1, 4 * D), lambda b, i, k, ir, gr: (0, 0)),
            pl.BlockSpec((4 * D, D), lambda b, i, k, ir, gr: (0, 0)),
            pl.BlockSpec((1, D), lambda b, i, k, ir, gr: (0, 0)),
            pl.BlockSpec((1, D, FFN),
                         lambda b, i, k, ir, gr: (ir[b, k], 0, 0)),
            pl.BlockSpec((1, 1, FFN),
                         lambda b, i, k, ir, gr: (ir[b, k], 0, 0)),
            pl.BlockSpec((1, FFN, D),
                         lambda b, i, k, ir, gr: (ir[b, k], 0, 0)),
            pl.BlockSpec((1, 1, D),
                         lambda b, i, k, ir, gr: (ir[b, k], 0, 0)),
        ],
        out_specs=pl.BlockSpec(
            (1, BS, D), lambda b, i, k, ir, gr: (b, i, 0)),
    )
    out_b = pl.pallas_call(
        _mlp_moe_kernel,
        grid_spec=grid_spec,
        out_shape=jax.ShapeDtypeStruct((B, S, D), _F32),
        compiler_params=pltpu.CompilerParams(
            dimension_semantics=seq * 3),
    )(idx_p, gate_p, x1, ln2_w.reshape(1, D), ln2_b.reshape(1, D), w_fc,
      c_fc_b.reshape(1, 4 * D), w_pr, c_proj_b.reshape(1, D),
      down_w, db2, up_w, ub2)

    return jnp.transpose(out_b, (1, 0, 2))


# speedup vs baseline: 1.0913x; 1.0179x over previous
import jax
import jax.numpy as jnp
from jax.experimental import pallas as pl
from jax.experimental.pallas import tpu as pltpu
from jax.experimental.pallas import tpu_sc as plsc

D = 768
H = 12
HD = 64
E = 64
K = 2
FFN = 64
SCALE = 0.1
NEG = -1e30

_F32 = jnp.float32
_BF16 = jnp.bfloat16


def _ln_qkv_kernel(x_ref, lnw_ref, lnb_ref, w_ref, b_ref, o_ref):
    x = x_ref[0].astype(_F32)
    m = jnp.mean(x, axis=1, keepdims=True)
    v = jnp.mean((x - m) ** 2, axis=1, keepdims=True)
    xn = (x - m) / jnp.sqrt(v + 1e-5) * lnw_ref[...] + lnb_ref[...]
    y = jnp.dot(xn.astype(_BF16), w_ref[...], preferred_element_type=_F32)
    y = y + b_ref[...]
    o_ref[0] = y.astype(_BF16)


def _attn_kernel(q_ref, k_ref, v_ref, x_ref, w_ref, b_ref, rw_ref,
                 o_ref, l_ref):
    b = pl.program_id(0)
    i = pl.program_id(1)
    hp = pl.program_id(2)
    q2 = q_ref[0]
    k2 = k_ref[0]
    v2 = v_ref[0]
    CK = 256
    S_FULL = k2.shape[0]
    outs = []
    for h in range(2):
        q = q2[:, h * HD:(h + 1) * HD]
        acc = None
        den = None
        for c in range(S_FULL // CK):
            kc = k2[c * CK:(c + 1) * CK, h * HD:(h + 1) * HD]
            vc = v2[c * CK:(c + 1) * CK, h * HD:(h + 1) * HD]
            s = jax.lax.dot_general(q, kc, (((1,), (1,)), ((), ())),
                                    preferred_element_type=_F32)
            e = jnp.exp(s)
            d = jnp.sum(e, axis=1, keepdims=True)
            o = jnp.dot(e.astype(_BF16), vc, preferred_element_type=_F32)
            acc = o if acc is None else acc + o
            den = d if den is None else den + d
        outs.append((acc * (1.0 / den)).astype(_BF16))
    o2 = jnp.concatenate(outs, axis=1)
    po = jnp.dot(o2, w_ref[...], preferred_element_type=_F32)

    @pl.when(hp == 0)
    def _():
        o_ref[0] = x_ref[0] + b_ref[...]

    o_ref[0] += po

    @pl.when(jnp.logical_and(i == 0, hp == H // 2 - 1))
    def _():
        row0 = o_ref[0][0:1, :]
        lb = jnp.dot(row0.astype(_BF16), rw_ref[...].astype(_BF16),
                     preferred_element_type=_F32)
        rows = jax.lax.broadcasted_iota(jnp.int32, (8, E), 0)

        @pl.when(b == 0)
        def _():
            l_ref[...] = jnp.where(rows == 0, lb, 0.0)

        @pl.when(b != 0)
        def _():
            l_ref[...] = jnp.where(rows == 1, lb, l_ref[...])


def _sc_router_body(l_hbm, idx_hbm, val_hbm, lbuf, ibuf, vbuf, sem):
    core = jax.lax.axis_index("core")

    @pl.when(core == 0)
    def _():
        pltpu.async_copy(l_hbm, lbuf, sem).wait()
        for b in range(2):
            vbuf[b, 0] = lbuf[b, 0]
            ibuf[b, 0] = 0

            @pl.loop(1, E)
            def _(i):
                v = lbuf[b, i]

                @pl.when(v > vbuf[b, 0])
                def _():
                    vbuf[b, 0] = v
                    ibuf[b, 0] = i

            vbuf[b, 1] = NEG
            ibuf[b, 1] = 0

            @pl.loop(0, E)
            def _(i):
                v = lbuf[b, i]

                @pl.when(jnp.logical_and(i != ibuf[b, 0], v > vbuf[b, 1]))
                def _():
                    vbuf[b, 1] = v
                    ibuf[b, 1] = i

        pltpu.async_copy(ibuf, idx_hbm, sem).wait()
        pltpu.async_copy(vbuf, val_hbm, sem).wait()


def _mlp_moe_kernel(idx_ref, g_ref, x_ref, lnw_ref, lnb_ref, wfc_ref, bfc_ref,
                    wpr_ref, bpr_ref, dw_ref, db_ref, uw_ref, ub_ref, o_ref):
    b = pl.program_id(0)
    k = pl.program_id(2)
    x = x_ref[0]

    @pl.when(k == 0)
    def _():
        m = jnp.mean(x, axis=1, keepdims=True)
        v = jnp.mean((x - m) ** 2, axis=1, keepdims=True)
        xn = (x - m) / jnp.sqrt(v + 1e-5) * lnw_ref[...] + lnb_ref[...]
        h = jnp.dot(xn.astype(_BF16), wfc_ref[...],
                    preferred_element_type=_F32)
        h = h + bfc_ref[...]
        h = h * jax.nn.sigmoid(1.702 * h)
        y = jnp.dot(h.astype(_BF16), wpr_ref[...],
                    preferred_element_type=_F32)
        o_ref[0] = y + bpr_ref[...] + x

    l1 = g_ref[b, 0]
    l2 = g_ref[b, 1]
    g1 = 1.0 / (1.0 + jnp.exp(l2 - l1))
    g = jnp.where(k == 0, g1, 1.0 - g1) * SCALE
    xh = x.astype(_BF16)
    hh = jnp.dot(xh, dw_ref[0].astype(_BF16),
                 preferred_element_type=_F32) + db_ref[0]
    hh = jnp.maximum(hh, 0.0)
    up = jnp.dot(hh.astype(_BF16), uw_ref[0].astype(_BF16),
                 preferred_element_type=_F32)
    o_ref[0] += g * (up + ub_ref[0])


def kernel(x, in_proj_w, in_proj_b, out_proj_w, out_proj_b, ln1_w, ln1_b,
           ln2_w, ln2_b, c_fc_w, c_fc_b, c_proj_w, c_proj_b, router,
           down_w, down_b, up_w, up_b):
    S, B, _ = x.shape
    BS = 1024
    BQ = 2048
    nS = S // BS

    xb = jnp.transpose(x, (1, 0, 2))
    qscale = jnp.concatenate([jnp.full((D,), 0.125, _F32),
                              jnp.ones((2 * D,), _F32)])
    w_in = (in_proj_w.T * qscale).astype(_BF16)
    in_proj_b = in_proj_b * qscale
    w_out = out_proj_w.T.astype(_BF16)
    w_fc = c_fc_w.T.astype(_BF16)
    w_pr = c_proj_w.T.astype(_BF16)
    db2 = down_b.reshape(E, 1, FFN)
    ub2 = up_b.reshape(E, 1, D)

    seq = ("arbitrary",)

    qkv = pl.pallas_call(
        _ln_qkv_kernel,
        grid=(B, nS),
        in_specs=[
            pl.BlockSpec((1, BS, D), lambda b, i: (b, i, 0)),
            pl.BlockSpec((1, D), lambda b, i: (0, 0)),
            pl.BlockSpec((1, D), lambda b, i: (0, 0)),
            pl.BlockSpec((D, 3 * D), lambda b, i: (0, 0)),
            pl.BlockSpec((1, 3 * D), lambda b, i: (0, 0)),
        ],
        out_specs=pl.BlockSpec((1, BS, 3 * D), lambda b, i: (b, i, 0)),
        out_shape=jax.ShapeDtypeStruct((B, S, 3 * D), _BF16),
        compiler_params=pltpu.CompilerParams(
            dimension_semantics=seq * 2),
    )(xb, ln1_w.reshape(1, D), ln1_b.reshape(1, D), w_in,
      in_proj_b.reshape(1, 3 * D))

    HP = H // 2
    x1, logits = pl.pallas_call(
        _attn_kernel,
        grid=(B, S // BQ, HP),
        in_specs=[
            pl.BlockSpec((1, BQ, 2 * HD), lambda b, i, h: (b, i, h)),
            pl.BlockSpec((1, S, 2 * HD), lambda b, i, h: (b, 0, HP + h)),
            pl.BlockSpec((1, S, 2 * HD), lambda b, i, h: (b, 0, 2 * HP + h)),
            pl.BlockSpec((1, BQ, D), lambda b, i, h: (b, i, 0)),
            pl.BlockSpec((2 * HD, D), lambda b, i, h: (h, 0)),
            pl.BlockSpec((1, D), lambda b, i, h: (0, 0)),
            pl.BlockSpec((D, E), lambda b, i, h: (0, 0)),
        ],
        out_specs=[
            pl.BlockSpec((1, BQ, D), lambda b, i, h: (b, i, 0)),
            pl.BlockSpec((8, E), lambda b, i, h: (0, 0)),
        ],
        out_shape=[
            jax.ShapeDtypeStruct((B, S, D), _F32),
            jax.ShapeDtypeStruct((8, E), _F32),
        ],
        compiler_params=pltpu.CompilerParams(
            dimension_semantics=seq * 3),
    )(qkv, qkv, qkv, xb, w_out, out_proj_b.reshape(1, D), router)

    sc_mesh = plsc.ScalarSubcoreMesh(axis_name="core", num_cores=2)
    idx_p, gate_p = pl.kernel(
        _sc_router_body,
        out_type=[
            jax.ShapeDtypeStruct((2, 16), jnp.int32),
            jax.ShapeDtypeStruct((2, 16), _F32),
        ],
        mesh=sc_mesh,
        scratch_types=[
            pltpu.SMEM((8, E), _F32),
            pltpu.SMEM((2, 16), jnp.int32),
            pltpu.SMEM((2, 16), _F32),
            pltpu.SemaphoreType.DMA,
        ],
    )(logits)

    grid_spec = pltpu.PrefetchScalarGridSpec(
        num_scalar_prefetch=2,
        grid=(B, nS, K),
        in_specs=[
            pl.BlockSpec((1, BS, D), lambda b, i, k, ir, gr: (b, i, 0)),
            pl.BlockSpec((1, D), lambda b, i, k, ir, gr: (0, 0)),
            pl.BlockSpec((1, D), lambda b, i, k, ir, gr: (0, 0)),
            pl.BlockSpec((D, 4 * D), lambda b, i, k, ir, gr: (0, 0)),
            pl.BlockSpec((1, 4 * D), lambda b, i, k, ir, gr: (0, 0)),
            pl.BlockSpec((4 * D, D), lambda b, i, k, ir, gr: (0, 0)),
            pl.BlockSpec((1, D), lambda b, i, k, ir, gr: (0, 0)),
            pl.BlockSpec((1, D, FFN),
                         lambda b, i, k, ir, gr: (ir[b, k], 0, 0)),
            pl.BlockSpec((1, 1, FFN),
                         lambda b, i, k, ir, gr: (ir[b, k], 0, 0)),
            pl.BlockSpec((1, FFN, D),
                         lambda b, i, k, ir, gr: (ir[b, k], 0, 0)),
            pl.BlockSpec((1, 1, D),
                         lambda b, i, k, ir, gr: (ir[b, k], 0, 0)),
        ],
        out_specs=pl.BlockSpec(
            (1, BS, D), lambda b, i, k, ir, gr: (b, i, 0)),
    )
    out_b = pl.pallas_call(
        _mlp_moe_kernel,
        grid_spec=grid_spec,
        out_shape=jax.ShapeDtypeStruct((B, S, D), _F32),
        compiler_params=pltpu.CompilerParams(
            dimension_semantics=seq * 3),
    )(idx_p, gate_p, x1, ln2_w.reshape(1, D), ln2_b.reshape(1, D), w_fc,
      c_fc_b.reshape(1, 4 * D), w_pr, c_proj_b.reshape(1, D),
      down_w, db2, up_w, ub2)

    return jnp.transpose(out_b, (1, 0, 2))
